# baseline (speedup 1.0000x reference)
import functools

import jax
import jax.numpy as jnp
from jax import lax
from jax.experimental import pallas as pl
from jax.experimental.pallas import tpu as pltpu
from jax.experimental.pallas import tpu_sc as plsc

_VOCAB = 100000
_DIM = 768
_SEQ = 8192

_NC = 2
_NS = 16
_NW = _NC * _NS
_ROWS_PER_W = _SEQ // _NW
_CHUNK = 8
_NCHUNKS = _ROWS_PER_W // _CHUNK
_NBUF = 4
_LANES = 16
_VECS_PER_ROW = _DIM // _LANES


def _embed_body(x_hbm, emb0_hbm, emb1_hbm, out_hbm, idx_v, tok_v, pos_v,
                sem_g, sem_p, sem_s):
    wid = lax.axis_index("s") * _NC + lax.axis_index("c")
    base = wid * _ROWS_PER_W

    pltpu.sync_copy(x_hbm.at[pl.ds(base, _ROWS_PER_W)], idx_v)

    def loads(c):
        b = c % _NBUF
        g = pltpu.make_async_copy(
            emb0_hbm.at[idx_v.at[pl.ds(c * _CHUNK, _CHUNK)]], tok_v.at[b],
            sem_g.at[b])
        p = pltpu.make_async_copy(
            emb1_hbm.at[pl.ds(base + c * _CHUNK, _CHUNK)], pos_v.at[b],
            sem_p.at[b])
        return g, p

    def store(c):
        b = c % _NBUF
        return pltpu.make_async_copy(
            pos_v.at[b], out_hbm.at[pl.ds(base + c * _CHUNK, _CHUNK)],
            sem_s.at[b])

    for f in range(_NBUF - 1):
        g, p = loads(f)
        g.start()
        p.start()

    for c in range(_NCHUNKS):
        b = c % _NBUF
        g, p = loads(c)
        g.wait()
        p.wait()

        def add_row(r, carry):
            for j in range(_VECS_PER_ROW):
                v = tok_v[b, r, pl.ds(j * _LANES, _LANES)]
                plsc.addupdate(pos_v.at[b, r, pl.ds(j * _LANES, _LANES)], v)
            return carry

        lax.fori_loop(0, _CHUNK, add_row, 0)

        store(c).start()
        f = c + _NBUF - 1
        if f < _NCHUNKS:
            if f - _NBUF >= 0:
                store(f - _NBUF).wait()
            g, p = loads(f)
            g.start()
            p.start()

    for c in range(_NCHUNKS - _NBUF, _NCHUNKS):
        store(c).wait()


@jax.jit
def _embed(x, emb0, emb1):
    mesh = plsc.VectorSubcoreMesh(core_axis_name="c", subcore_axis_name="s")
    run = functools.partial(
        pl.kernel,
        out_type=jax.ShapeDtypeStruct((_SEQ, _DIM), jnp.float32),
        mesh=mesh,
        scratch_types=[
            pltpu.VMEM((_ROWS_PER_W,), jnp.int32),
            pltpu.VMEM((_NBUF, _CHUNK, _DIM), jnp.float32),
            pltpu.VMEM((_NBUF, _CHUNK, _DIM), jnp.float32),
            pltpu.SemaphoreType.DMA((_NBUF,)),
            pltpu.SemaphoreType.DMA((_NBUF,)),
            pltpu.SemaphoreType.DMA((_NBUF,)),
        ],
    )(_embed_body)
    return run(x, emb0, emb1)


def kernel(x, emb0, emb1):
    return _embed(x, emb0, emb1)


# speedup vs baseline: 1.0039x; 1.0039x over previous
import functools

import jax
import jax.numpy as jnp
from jax import lax
from jax.experimental import pallas as pl
from jax.experimental.pallas import tpu as pltpu
from jax.experimental.pallas import tpu_sc as plsc

_VOCAB = 100000
_DIM = 768
_SEQ = 8192

_NC = 2
_NS = 16
_NW = _NC * _NS
_ROWS_PER_W = _SEQ // _NW
_CHUNK = 8
_NCHUNKS = _ROWS_PER_W // _CHUNK
_NBUF = 8
_LANES = 16
_VECS_PER_ROW = _DIM // _LANES


def _embed_body(x_hbm, emb0_hbm, emb1_hbm, out_hbm, idx_v, tok_v, pos_v,
                sem_g, sem_p, sem_s):
    wid = lax.axis_index("s") * _NC + lax.axis_index("c")
    base = wid * _ROWS_PER_W

    pltpu.sync_copy(x_hbm.at[pl.ds(base, _ROWS_PER_W)], idx_v)

    def loads(c):
        b = c % _NBUF
        g = pltpu.make_async_copy(
            emb0_hbm.at[idx_v.at[pl.ds(c * _CHUNK, _CHUNK)]], tok_v.at[b],
            sem_g.at[b])
        p = pltpu.make_async_copy(
            emb1_hbm.at[pl.ds(base + c * _CHUNK, _CHUNK)], pos_v.at[b],
            sem_p.at[b])
        return g, p

    def store(c):
        b = c % _NBUF
        return pltpu.make_async_copy(
            pos_v.at[b], out_hbm.at[pl.ds(base + c * _CHUNK, _CHUNK)],
            sem_s.at[b])

    for f in range(_NBUF - 1):
        g, p = loads(f)
        g.start()
        p.start()

    for c in range(_NCHUNKS):
        b = c % _NBUF
        g, p = loads(c)
        g.wait()
        p.wait()

        def add_row(r, carry):
            for j in range(_VECS_PER_ROW):
                v = tok_v[b, r, pl.ds(j * _LANES, _LANES)]
                plsc.addupdate(pos_v.at[b, r, pl.ds(j * _LANES, _LANES)], v)
            return carry

        lax.fori_loop(0, _CHUNK, add_row, 0)

        store(c).start()
        f = c + _NBUF - 1
        if f < _NCHUNKS:
            if f - _NBUF >= 0:
                store(f - _NBUF).wait()
            g, p = loads(f)
            g.start()
            p.start()

    for c in range(_NCHUNKS - _NBUF, _NCHUNKS):
        store(c).wait()


@jax.jit
def _embed(x, emb0, emb1):
    mesh = plsc.VectorSubcoreMesh(core_axis_name="c", subcore_axis_name="s")
    run = functools.partial(
        pl.kernel,
        out_type=jax.ShapeDtypeStruct((_SEQ, _DIM), jnp.float32),
        mesh=mesh,
        scratch_types=[
            pltpu.VMEM((_ROWS_PER_W,), jnp.int32),
            pltpu.VMEM((_NBUF, _CHUNK, _DIM), jnp.float32),
            pltpu.VMEM((_NBUF, _CHUNK, _DIM), jnp.float32),
            pltpu.SemaphoreType.DMA((_NBUF,)),
            pltpu.SemaphoreType.DMA((_NBUF,)),
            pltpu.SemaphoreType.DMA((_NBUF,)),
        ],
    )(_embed_body)
    return run(x, emb0, emb1)


def kernel(x, emb0, emb1):
    return _embed(x, emb0, emb1)


# speedup vs baseline: 1.0092x; 1.0053x over previous
import functools

import jax
import jax.numpy as jnp
from jax import lax
from jax.experimental import pallas as pl
from jax.experimental.pallas import tpu as pltpu
from jax.experimental.pallas import tpu_sc as plsc

_VOCAB = 100000
_DIM = 768
_SEQ = 8192

_NC = 2
_NS = 16
_NW = _NC * _NS
_ROWS_PER_W = _SEQ // _NW
_CHUNK = 8
_NCHUNKS = _ROWS_PER_W // _CHUNK
_NBUF = 6
_LANES = 16
_VECS_PER_ROW = _DIM // _LANES


def _embed_body(x_hbm, emb0_hbm, emb1_hbm, out_hbm, idx_v, tok_v, pos_v,
                sem_g, sem_p, sem_s):
    wid = lax.axis_index("s") * _NC + lax.axis_index("c")
    base = wid * _ROWS_PER_W

    pltpu.sync_copy(x_hbm.at[pl.ds(base, _ROWS_PER_W)], idx_v)

    def loads(c):
        b = c % _NBUF
        g = pltpu.make_async_copy(
            emb0_hbm.at[idx_v.at[pl.ds(c * _CHUNK, _CHUNK)]], tok_v.at[b],
            sem_g.at[b])
        p = pltpu.make_async_copy(
            emb1_hbm.at[pl.ds(base + c * _CHUNK, _CHUNK)], pos_v.at[b],
            sem_p.at[b])
        return g, p

    def store(c):
        b = c % _NBUF
        return pltpu.make_async_copy(
            pos_v.at[b], out_hbm.at[pl.ds(base + c * _CHUNK, _CHUNK)],
            sem_s.at[b])

    for f in range(_NBUF - 1):
        g, p = loads(f)
        g.start()
        p.start()

    for c in range(_NCHUNKS):
        b = c % _NBUF
        g, p = loads(c)
        g.wait()
        p.wait()

        def add_row(r, carry):
            for j in range(_VECS_PER_ROW):
                v = tok_v[b, r, pl.ds(j * _LANES, _LANES)]
                plsc.addupdate(pos_v.at[b, r, pl.ds(j * _LANES, _LANES)], v)
            return carry

        lax.fori_loop(0, _CHUNK, add_row, 0)

        store(c).start()
        f = c + _NBUF - 1
        if f < _NCHUNKS:
            if f - _NBUF >= 0:
                store(f - _NBUF).wait()
            g, p = loads(f)
            g.start()
            p.start()

    for c in range(_NCHUNKS - _NBUF, _NCHUNKS):
        store(c).wait()


@jax.jit
def _embed(x, emb0, emb1):
    mesh = plsc.VectorSubcoreMesh(core_axis_name="c", subcore_axis_name="s")
    run = functools.partial(
        pl.kernel,
        out_type=jax.ShapeDtypeStruct((_SEQ, _DIM), jnp.float32),
        mesh=mesh,
        scratch_types=[
            pltpu.VMEM((_ROWS_PER_W,), jnp.int32),
            pltpu.VMEM((_NBUF, _CHUNK, _DIM), jnp.float32),
            pltpu.VMEM((_NBUF, _CHUNK, _DIM), jnp.float32),
            pltpu.SemaphoreType.DMA((_NBUF,)),
            pltpu.SemaphoreType.DMA((_NBUF,)),
            pltpu.SemaphoreType.DMA((_NBUF,)),
        ],
    )(_embed_body)
    return run(x, emb0, emb1)


def kernel(x, emb0, emb1):
    return _embed(x, emb0, emb1)


# speedup vs baseline: 1.1611x; 1.1506x over previous
import functools

import jax
import jax.numpy as jnp
from jax import lax
from jax.experimental import pallas as pl
from jax.experimental.pallas import tpu as pltpu
from jax.experimental.pallas import tpu_sc as plsc

_VOCAB = 100000
_DIM = 768
_SEQ = 8192

_NC = 2
_NS = 16
_NW = _NC * _NS
_ROWS_PER_W = _SEQ // _NW
_CHUNK = 8
_NCHUNKS = _ROWS_PER_W // _CHUNK
_NBUF = 8
_LANES = 16
_VECS_PER_ROW = _DIM // _LANES


def _embed_body(x_hbm, emb0_hbm, emb1_hbm, out_hbm, idx_v, tok_v, pos_v,
                sem_g, sem_p, sem_s):
    wid = lax.axis_index("s") * _NC + lax.axis_index("c")
    base = wid * _ROWS_PER_W

    pltpu.sync_copy(x_hbm.at[pl.ds(base, _ROWS_PER_W)], idx_v)

    def loads(c, b):
        g = pltpu.make_async_copy(
            emb0_hbm.at[idx_v.at[pl.ds(c * _CHUNK, _CHUNK)]], tok_v.at[b],
            sem_g.at[b])
        p = pltpu.make_async_copy(
            emb1_hbm.at[pl.ds(base + c * _CHUNK, _CHUNK)], pos_v.at[b],
            sem_p.at[b])
        return g, p

    def store(c, b):
        return pltpu.make_async_copy(
            pos_v.at[b], out_hbm.at[pl.ds(base + c * _CHUNK, _CHUNK)],
            sem_s.at[b])

    for f in range(_NBUF - 1):
        g, p = loads(f, f)
        g.start()
        p.start()

    def group(gi, carry):
        for b in range(_NBUF):
            c = gi * _NBUF + b
            g, p = loads(c, b)
            g.wait()
            p.wait()

            def add_row(r, cc):
                for j in range(_VECS_PER_ROW):
                    v = tok_v[b, r, pl.ds(j * _LANES, _LANES)]
                    plsc.addupdate(pos_v.at[b, r, pl.ds(j * _LANES, _LANES)],
                                   v)
                return cc

            lax.fori_loop(0, _CHUNK, add_row, 0)

            store(c, b).start()
            f = c + _NBUF - 1
            fslot = (b + _NBUF - 1) % _NBUF

            @pl.when(f < _NCHUNKS)
            def _issue():
                @pl.when(f >= _NBUF)
                def _wait_prev_store():
                    store(f - _NBUF, fslot).wait()
                g2, p2 = loads(f, fslot)
                g2.start()
                p2.start()

        return carry

    lax.fori_loop(0, _NCHUNKS // _NBUF, group, 0)

    for i in range(_NBUF):
        c = _NCHUNKS - _NBUF + i
        store(c, c % _NBUF).wait()


@jax.jit
def _embed(x, emb0, emb1):
    mesh = plsc.VectorSubcoreMesh(core_axis_name="c", subcore_axis_name="s")
    run = functools.partial(
        pl.kernel,
        out_type=jax.ShapeDtypeStruct((_SEQ, _DIM), jnp.float32),
        mesh=mesh,
        scratch_types=[
            pltpu.VMEM((_ROWS_PER_W,), jnp.int32),
            pltpu.VMEM((_NBUF, _CHUNK, _DIM), jnp.float32),
            pltpu.VMEM((_NBUF, _CHUNK, _DIM), jnp.float32),
            pltpu.SemaphoreType.DMA((_NBUF,)),
            pltpu.SemaphoreType.DMA((_NBUF,)),
            pltpu.SemaphoreType.DMA((_NBUF,)),
        ],
    )(_embed_body)
    return run(x, emb0, emb1)


def kernel(x, emb0, emb1):
    return _embed(x, emb0, emb1)


# speedup vs baseline: 1.2086x; 1.0408x over previous
import functools

import jax
import jax.numpy as jnp
from jax import lax
from jax.experimental import pallas as pl
from jax.experimental.pallas import tpu as pltpu
from jax.experimental.pallas import tpu_sc as plsc

_VOCAB = 100000
_DIM = 768
_SEQ = 8192

_NC = 2
_NS = 16
_NW = _NC * _NS
_ROWS_PER_W = _SEQ // _NW
_CHUNK = 8
_NCHUNKS = _ROWS_PER_W // _CHUNK
_NBUF = 4
_LANES = 16
_VECS_PER_ROW = _DIM // _LANES


def _embed_body(x_hbm, emb0_hbm, emb1_hbm, out_hbm, idx_v, tok_v, pos_v,
                sem_g, sem_p, sem_s):
    wid = lax.axis_index("s") * _NC + lax.axis_index("c")
    base = wid * _ROWS_PER_W

    pltpu.sync_copy(x_hbm.at[pl.ds(base, _ROWS_PER_W)], idx_v)

    def loads(c, b):
        g = pltpu.make_async_copy(
            emb0_hbm.at[idx_v.at[pl.ds(c * _CHUNK, _CHUNK)]], tok_v.at[b],
            sem_g.at[b])
        p = pltpu.make_async_copy(
            emb1_hbm.at[pl.ds(base + c * _CHUNK, _CHUNK)], pos_v.at[b],
            sem_p.at[b])
        return g, p

    def store(c, b):
        return pltpu.make_async_copy(
            pos_v.at[b], out_hbm.at[pl.ds(base + c * _CHUNK, _CHUNK)],
            sem_s.at[b])

    for f in range(_NBUF - 1):
        g, p = loads(f, f)
        g.start()
        p.start()

    def group(gi, carry):
        for b in range(_NBUF):
            c = gi * _NBUF + b
            g, p = loads(c, b)
            g.wait()
            p.wait()

            def add_row(r, cc):
                for j in range(_VECS_PER_ROW):
                    v = tok_v[b, r, pl.ds(j * _LANES, _LANES)]
                    plsc.addupdate(pos_v.at[b, r, pl.ds(j * _LANES, _LANES)],
                                   v)
                return cc

            lax.fori_loop(0, _CHUNK, add_row, 0)

            store(c, b).start()
            f = c + _NBUF - 1
            fslot = (b + _NBUF - 1) % _NBUF

            @pl.when(f < _NCHUNKS)
            def _issue():
                @pl.when(f >= _NBUF)
                def _wait_prev_store():
                    store(f - _NBUF, fslot).wait()
                g2, p2 = loads(f, fslot)
                g2.start()
                p2.start()

        return carry

    lax.fori_loop(0, _NCHUNKS // _NBUF, group, 0)

    for i in range(_NBUF):
        c = _NCHUNKS - _NBUF + i
        store(c, c % _NBUF).wait()


@jax.jit
def _embed(x, emb0, emb1):
    mesh = plsc.VectorSubcoreMesh(core_axis_name="c", subcore_axis_name="s")
    run = functools.partial(
        pl.kernel,
        out_type=jax.ShapeDtypeStruct((_SEQ, _DIM), jnp.float32),
        mesh=mesh,
        scratch_types=[
            pltpu.VMEM((_ROWS_PER_W,), jnp.int32),
            pltpu.VMEM((_NBUF, _CHUNK, _DIM), jnp.float32),
            pltpu.VMEM((_NBUF, _CHUNK, _DIM), jnp.float32),
            pltpu.SemaphoreType.DMA((_NBUF,)),
            pltpu.SemaphoreType.DMA((_NBUF,)),
            pltpu.SemaphoreType.DMA((_NBUF,)),
        ],
    )(_embed_body)
    return run(x, emb0, emb1)


def kernel(x, emb0, emb1):
    return _embed(x, emb0, emb1)
